# baseline (speedup 1.0000x reference)
import functools

import jax
import jax.numpy as jnp
from jax import lax
from jax.experimental import pallas as pl
from jax.experimental.pallas import tpu as pltpu
from jax.experimental.pallas import tpu_sc as plsc

N = 5000
NP = 5120
TJ = 512
TI = 512
IOU_THRESHOLD = 0.5
SCORE_THRESHOLD = 0.05


def _nms_kernel(x0r, y0r, x1r, y1r,
                x0c, y0c, x1c, y1c, sc,
                ox0, oy0, ox1, oy1, osc):
    jt = pl.program_id(0)
    j0 = jt * TJ

    gj = j0 + lax.broadcasted_iota(jnp.int32, (1, TJ), 1)
    cx0 = x0c[...]
    cy0 = y0c[...]
    cx1 = x1c[...]
    cy1 = y1c[...]
    area_c = (cx1 - cx0) * (cy1 - cy0)

    def iou_tile(i0):
        rx0 = x0r[pl.ds(i0, TI), :]
        ry0 = y0r[pl.ds(i0, TI), :]
        rx1 = x1r[pl.ds(i0, TI), :]
        ry1 = y1r[pl.ds(i0, TI), :]
        area_r = (rx1 - rx0) * (ry1 - ry0)

        ltx = jnp.maximum(rx0, cx0)
        lty = jnp.maximum(ry0, cy0)
        rbx = jnp.minimum(rx1, cx1)
        rby = jnp.minimum(ry1, cy1)
        w = jnp.maximum(rbx - ltx, 0.0)
        h = jnp.maximum(rby - lty, 0.0)
        inter = w * h
        union = area_r + area_c - inter
        return inter / union

    def pmax8(i0):
        return jnp.max(iou_tile(i0).reshape(TI // 8, 8, TJ), axis=0)

    def body2(it, acc8):
        a = pmax8((2 * it) * TI)
        b = pmax8((2 * it + 1) * TI)
        return jnp.maximum(acc8, jnp.maximum(a, b))

    acc8 = lax.fori_loop(0, jt // 2, body2, jnp.zeros((8, TJ), jnp.float32))
    acc8 = lax.cond(jt % 2 == 1,
                    lambda a: jnp.maximum(a, pmax8((jt - 1) * TI)),
                    lambda a: a, acc8)

    i0 = jt * TJ
    gi = i0 + lax.broadcasted_iota(jnp.int32, (TI, 1), 0)
    diag = jnp.where(gi < gj, iou_tile(i0), 0.0)
    acc8 = jnp.maximum(acc8, jnp.max(diag.reshape(TI // 8, 8, TJ), axis=0))
    acc = jnp.max(acc8, axis=0, keepdims=True)

    s = sc[...]
    keep = (acc <= IOU_THRESHOLD) & (s > SCORE_THRESHOLD)
    m = keep.astype(jnp.float32)
    ox0[...] = cx0 * m
    oy0[...] = cy0 * m
    ox1[...] = cx1 * m
    oy1[...] = cy1 * m
    osc[...] = s * m


_SC_MESH = plsc.VectorSubcoreMesh(core_axis_name="c", subcore_axis_name="s")


@functools.partial(
    pl.kernel,
    mesh=_SC_MESH,
    out_type=jax.ShapeDtypeStruct((32, 16), jnp.float32),
    scratch_types=[pltpu.VMEM((16,), jnp.float32)],
)
def _sc_probe(out_hbm, buf):
    wid = lax.axis_index("s") * 2 + lax.axis_index("c")

    def body(i, acc):
        return acc * 1.0000001 + 1.0

    acc = lax.fori_loop(0, 4000, body, jnp.zeros((16,), jnp.float32))
    buf[...] = acc
    pltpu.sync_copy(buf, out_hbm.at[wid])


@jax.jit
def kernel(boxes, scores):
    junk = _sc_probe()
    _, x0, y0, x1, y1, s = lax.sort(
        (-scores, boxes[:, 0], boxes[:, 1], boxes[:, 2], boxes[:, 3], scores),
        num_keys=1)

    pad = NP - N
    cols = [jnp.pad(c, ((0, pad),)) for c in (x0, y0, x1, y1)]
    scol = jnp.pad(s, ((0, pad),)).reshape(1, NP)
    rows = [c.reshape(NP, 1) for c in cols]
    cols = [c.reshape(1, NP) for c in cols]

    row_spec = pl.BlockSpec((NP, 1), lambda j: (0, 0))
    col_spec = pl.BlockSpec((1, TJ), lambda j: (0, j))

    outs = pl.pallas_call(
        _nms_kernel,
        grid=(NP // TJ,),
        in_specs=[row_spec] * 4 + [col_spec] * 5,
        out_specs=[col_spec] * 5,
        out_shape=[jax.ShapeDtypeStruct((1, NP), jnp.float32)] * 5,
    )(*rows, *cols, scol)

    out = jnp.concatenate([o.reshape(NP, 1) for o in outs], axis=1)
    return out[:N] + junk[0, 0] * 0.0


# speedup vs baseline: 1.3431x; 1.3431x over previous
import functools

import jax
import jax.numpy as jnp
from jax import lax
from jax.experimental import pallas as pl
from jax.experimental.pallas import tpu as pltpu
from jax.experimental.pallas import tpu_sc as plsc

N = 5000
NP = 5120
TJ = 512
TI = 512
IOU_THRESHOLD = 0.5
SCORE_THRESHOLD = 0.05


def _nms_kernel(x0r, y0r, x1r, y1r,
                x0c, y0c, x1c, y1c, sc,
                ox0, oy0, ox1, oy1, osc):
    jt = pl.program_id(0)
    j0 = jt * TJ

    gj = j0 + lax.broadcasted_iota(jnp.int32, (1, TJ), 1)
    cx0 = x0c[...]
    cy0 = y0c[...]
    cx1 = x1c[...]
    cy1 = y1c[...]
    area_c = (cx1 - cx0) * (cy1 - cy0)

    def iou_tile(i0):
        rx0 = x0r[pl.ds(i0, TI), :]
        ry0 = y0r[pl.ds(i0, TI), :]
        rx1 = x1r[pl.ds(i0, TI), :]
        ry1 = y1r[pl.ds(i0, TI), :]
        area_r = (rx1 - rx0) * (ry1 - ry0)

        ltx = jnp.maximum(rx0, cx0)
        lty = jnp.maximum(ry0, cy0)
        rbx = jnp.minimum(rx1, cx1)
        rby = jnp.minimum(ry1, cy1)
        w = jnp.maximum(rbx - ltx, 0.0)
        h = jnp.maximum(rby - lty, 0.0)
        inter = w * h
        union = area_r + area_c - inter
        return inter / union

    def pmax8(i0):
        return jnp.max(iou_tile(i0).reshape(TI // 8, 8, TJ), axis=0)

    def body2(it, acc8):
        a = pmax8((2 * it) * TI)
        b = pmax8((2 * it + 1) * TI)
        return jnp.maximum(acc8, jnp.maximum(a, b))

    acc8 = lax.fori_loop(0, jt // 2, body2, jnp.zeros((8, TJ), jnp.float32))
    acc8 = lax.cond(jt % 2 == 1,
                    lambda a: jnp.maximum(a, pmax8((jt - 1) * TI)),
                    lambda a: a, acc8)

    i0 = jt * TJ
    gi = i0 + lax.broadcasted_iota(jnp.int32, (TI, 1), 0)
    diag = jnp.where(gi < gj, iou_tile(i0), 0.0)
    acc8 = jnp.maximum(acc8, jnp.max(diag.reshape(TI // 8, 8, TJ), axis=0))
    acc = jnp.max(acc8, axis=0, keepdims=True)

    s = sc[...]
    keep = (acc <= IOU_THRESHOLD) & (s > SCORE_THRESHOLD)
    m = keep.astype(jnp.float32)
    ox0[...] = cx0 * m
    oy0[...] = cy0 * m
    ox1[...] = cx1 * m
    oy1[...] = cy1 * m
    osc[...] = s * m


_SC_MESH = plsc.VectorSubcoreMesh(core_axis_name="c", subcore_axis_name="s")


@functools.partial(
    pl.kernel,
    mesh=_SC_MESH,
    out_type=jax.ShapeDtypeStruct((32, 16), jnp.float32),
    scratch_types=[pltpu.VMEM((16,), jnp.float32)],
)
def _sc_probe(out_hbm, buf):
    wid = lax.axis_index("s") * 2 + lax.axis_index("c")

    def body(i, acc):
        return acc * 1.0000001 + 1.0

    acc = lax.fori_loop(0, 4000, body, jnp.zeros((16,), jnp.float32))
    buf[...] = acc
    pltpu.sync_copy(buf, out_hbm.at[wid])


@jax.jit
def kernel(boxes, scores):
    negs, x0, y0, x1, y1 = lax.sort(
        (-scores, boxes[:, 0], boxes[:, 1], boxes[:, 2], boxes[:, 3]),
        num_keys=1)
    s = -negs

    pad = NP - N
    cols = [jnp.pad(c, ((0, pad),)) for c in (x0, y0, x1, y1)]
    scol = jnp.pad(s, ((0, pad),)).reshape(1, NP)
    rows = [c.reshape(NP, 1) for c in cols]
    cols = [c.reshape(1, NP) for c in cols]

    row_spec = pl.BlockSpec((NP, 1), lambda j: (0, 0))
    col_spec = pl.BlockSpec((1, TJ), lambda j: (0, j))

    outs = pl.pallas_call(
        _nms_kernel,
        grid=(NP // TJ,),
        in_specs=[row_spec] * 4 + [col_spec] * 5,
        out_specs=[col_spec] * 5,
        out_shape=[jax.ShapeDtypeStruct((1, NP), jnp.float32)] * 5,
    )(*rows, *cols, scol)

    out = jnp.concatenate([o.reshape(NP, 1) for o in outs], axis=1)
    return out[:N]
